# baseline (speedup 1.0000x reference)
import functools
import jax
import jax.numpy as jnp
from jax import lax
from jax.experimental import pallas as pl
from jax.experimental.pallas import tpu as pltpu
from jax.experimental.pallas import tpu_sc as plsc

_EPS = 1e-5
_BE = 6400
_CB = 1280


def _segmax_sc(z2t, dst, n_nodes):
    H, E = z2t.shape
    e_half = E // 2
    n_chunks = e_half // _CB
    gpairs = _CB // 32
    mesh = plsc.VectorSubcoreMesh(core_axis_name="c", subcore_axis_name="s")

    @functools.partial(
        pl.kernel,
        mesh=mesh,
        out_type=jax.ShapeDtypeStruct((2 * H, n_nodes), jnp.float32),
        compiler_params=pltpu.CompilerParams(needs_layout_passes=False),
        scratch_types=[
            pltpu.VMEM((8, n_nodes), jnp.float32),
            pltpu.VMEM((2, _CB), jnp.int32),
            pltpu.VMEM((2, 8, _CB), jnp.float32),
            pltpu.VMEM((n_nodes,), jnp.float32),
            pltpu.SemaphoreType.DMA,
            pltpu.SemaphoreType.DMA,
        ],
    )
    def segmax(z2t_hbm, dst_hbm, out_hbm, acc_v, idx_v, buf_v, arb_v,
               sem0, sem1):
        half = lax.axis_index("c")
        r0 = pl.multiple_of(lax.axis_index("s") * 8, 8)
        e0 = pl.multiple_of(half * e_half, 128)
        lanes0 = lax.iota(jnp.int32, 16).astype(jnp.float32)
        lanes1 = lanes0 + 16.0
        full = jnp.full((16,), True)
        neg_inf = jnp.full((16,), -jnp.inf, jnp.float32)
        sems = (sem0, sem1)

        def start(ch, b):
            cb = pl.multiple_of(e0 + ch * _CB, 128)
            pltpu.async_copy(dst_hbm.at[pl.ds(cb, _CB)], idx_v.at[b], sems[b])
            pltpu.async_copy(z2t_hbm.at[pl.ds(r0, 8), pl.ds(cb, _CB)],
                             buf_v.at[b], sems[b])

        def drain(b):
            pltpu.make_async_copy(dst_hbm.at[pl.ds(0, _CB)], idx_v.at[b],
                                  sems[b]).wait()
            pltpu.make_async_copy(z2t_hbm.at[pl.ds(0, 8), pl.ds(0, _CB)],
                                  buf_v.at[b], sems[b]).wait()

        start(0, 0)
        start(1, 1)

        def init_body(i, _):
            for c in range(8):
                acc_v[c, pl.ds(i * 16, 16)] = neg_inf
            return 0
        lax.fori_loop(0, n_nodes // 16, init_body, 0)

        def upd8(idx, b, g, mask):
            cvecs = [jnp.full((16,), c, jnp.int32) for c in range(8)]
            curs = [plsc.load_gather(acc_v, [cvecs[c], idx])
                    for c in range(8)]
            vs = [buf_v[b, c, pl.ds(g * 16, 16)] for c in range(8)]
            ms = [jnp.maximum(curs[c], vs[c]) for c in range(8)]
            for c in range(8):
                plsc.store_scatter(acc_v, [cvecs[c], idx], ms[c],
                                   mask=mask)

        def process(b):
            U = 4
            lanes = [lanes0 + (16.0 * u) for u in range(U)]

            def gq_body(g, _):
                idxs = [idx_v[b, pl.ds(g * (16 * U) + 16 * u, 16)]
                        for u in range(U)]
                for u in range(U):
                    plsc.store_scatter(arb_v, [idxs[u]], lanes[u], mask=full)
                wons = [plsc.load_gather(arb_v, [idxs[u]]) == lanes[u]
                        for u in range(U)]
                for u in range(U):
                    upd8(idxs[u], b, U * g + u, wons[u])

                def any_losers(carry):
                    out = jnp.any(carry[0])
                    for u in range(1, U):
                        out = out | jnp.any(carry[u])
                    return out

                def retry(carry):
                    for u in range(U):
                        plsc.store_scatter(arb_v, [idxs[u]], lanes[u],
                                           mask=carry[u])
                    ws = [(plsc.load_gather(arb_v, [idxs[u]]) == lanes[u])
                          & carry[u] for u in range(U)]
                    for u in range(U):
                        upd8(idxs[u], b, U * g + u, ws[u])
                    return tuple(carry[u] & jnp.logical_not(ws[u])
                                 for u in range(U))

                lax.while_loop(any_losers, retry,
                               tuple(jnp.logical_not(w) for w in wons))
                return 0
            lax.fori_loop(0, _CB // (16 * U), gq_body, 0)

        def pair_body(j, _):
            drain(0)
            process(0)

            @pl.when(2 * j + 2 < n_chunks)
            def _():
                start(2 * j + 2, 0)
            drain(1)
            process(1)

            @pl.when(2 * j + 3 < n_chunks)
            def _():
                start(2 * j + 3, 1)
            return 0
        lax.fori_loop(0, n_chunks // 2, pair_body, 0)
        if n_chunks % 2:
            drain(0)
            process(0)
        out_r = pl.multiple_of(half * H + r0, 8)
        pltpu.sync_copy(acc_v, out_hbm.at[pl.ds(out_r, 8), :])

    part = segmax(z2t, dst)
    return jnp.maximum(part[:H], part[H:])


_CG = 640


def _gather2_sc(x, src, dst):
    N, F = x.shape
    E = src.shape[0]
    n_chunks = E // _CG
    mesh = plsc.VectorSubcoreMesh(core_axis_name="c", subcore_axis_name="s")

    @functools.partial(
        pl.kernel,
        mesh=mesh,
        out_type=[jax.ShapeDtypeStruct((E, F), jnp.float32),
                  jax.ShapeDtypeStruct((E, F), jnp.float32)],
        compiler_params=pltpu.CompilerParams(needs_layout_passes=False),
        scratch_types=[
            pltpu.VMEM((_CG,), jnp.int32),
            pltpu.VMEM((_CG, 128), jnp.float32),
            pltpu.SemaphoreType.DMA,
        ],
    )
    def gather2(x_hbm, src_hbm, dst_hbm, xd_hbm, xs_hbm, idx_v, buf_v, sem):
        wid = lax.axis_index("s") * 2 + lax.axis_index("c")
        per_w = (n_chunks + 31) // 32

        def run(idx_hbm, out_hbm):
            def chunk_body(j, _):
                ch = wid + j * 32

                @pl.when(ch < n_chunks)
                def _():
                    cb = pl.multiple_of(ch * _CG, 128)
                    pltpu.sync_copy(idx_hbm.at[pl.ds(cb, _CG)], idx_v)
                    copies = [
                        pltpu.async_copy(
                            x_hbm.at[idx_v.at[pl.ds(k * 128, 128)]],
                            buf_v.at[pl.ds(k * 128, 128)], sem)
                        for k in range(_CG // 128)
                    ]
                    for c in copies:
                        c.wait()
                    pltpu.sync_copy(buf_v, out_hbm.at[pl.ds(cb, _CG), :])
                return 0
            lax.fori_loop(0, per_w, chunk_body, 0)

        run(dst_hbm, xd_hbm)
        run(src_hbm, xs_hbm)

    return gather2(x, src, dst)


def _mm1_body(xd_ref, xs_ref, w_ref, b_ref, z1_ref, st_ref):
    xd = xd_ref[...]
    m = jnp.concatenate([xd, xs_ref[...] - xd], axis=1)
    z1 = jnp.dot(m, w_ref[...], preferred_element_type=jnp.float32)
    z1 = jnp.maximum(z1 + b_ref[...], 0.0)
    z1_ref[...] = z1
    @pl.when(pl.program_id(0) == 0)
    def _():
        st_ref[...] = jnp.zeros_like(st_ref)
    st_ref[0:1, :] += jnp.sum(z1, axis=0, keepdims=True)
    st_ref[1:2, :] += jnp.sum(z1 * z1, axis=0, keepdims=True)


def _mm1(xd, xs, w1_t, b1):
    E, F = xd.shape
    H = w1_t.shape[1]
    z1, st = pl.pallas_call(
        _mm1_body,
        grid=(E // _BE,),
        in_specs=[
            pl.BlockSpec((_BE, F), lambda i: (i, 0)),
            pl.BlockSpec((_BE, F), lambda i: (i, 0)),
            pl.BlockSpec((2 * F, H), lambda i: (0, 0)),
            pl.BlockSpec((1, H), lambda i: (0, 0)),
        ],
        out_specs=[
            pl.BlockSpec((_BE, H), lambda i: (i, 0)),
            pl.BlockSpec((8, H), lambda i: (0, 0)),
        ],
        out_shape=[
            jax.ShapeDtypeStruct((E, H), jnp.float32),
            jax.ShapeDtypeStruct((8, H), jnp.float32),
        ],
    )(xd, xs, w1_t, b1.reshape(1, H))
    return z1, st[0], st[1]


def _mm2_body(z1_ref, g_ref, mu_ref, den_ref, be_ref, w_ref, b_ref, z2t_ref, st_ref):
    h1 = g_ref[...] * (z1_ref[...] - mu_ref[...]) / den_ref[...] + be_ref[...]
    z2t = lax.dot_general(w_ref[...], h1, (((1,), (1,)), ((), ())),
                          preferred_element_type=jnp.float32)
    z2t = jnp.maximum(z2t + b_ref[...], 0.0)
    z2t_ref[...] = z2t
    @pl.when(pl.program_id(0) == 0)
    def _():
        st_ref[...] = jnp.zeros_like(st_ref)
    st_ref[:, 0:1] += jnp.sum(z2t, axis=1, keepdims=True)
    st_ref[:, 1:2] += jnp.sum(z2t * z2t, axis=1, keepdims=True)


def _mm2(z1, g1, mu1, den1, be1, w2, b2):
    E, H = z1.shape
    z2t, st = pl.pallas_call(
        _mm2_body,
        grid=(E // _BE,),
        in_specs=[
            pl.BlockSpec((_BE, H), lambda i: (i, 0)),
            pl.BlockSpec((1, H), lambda i: (0, 0)),
            pl.BlockSpec((1, H), lambda i: (0, 0)),
            pl.BlockSpec((1, H), lambda i: (0, 0)),
            pl.BlockSpec((1, H), lambda i: (0, 0)),
            pl.BlockSpec((H, H), lambda i: (0, 0)),
            pl.BlockSpec((H, 1), lambda i: (0, 0)),
        ],
        out_specs=[
            pl.BlockSpec((H, _BE), lambda i: (0, i)),
            pl.BlockSpec((H, 8), lambda i: (0, 0)),
        ],
        out_shape=[
            jax.ShapeDtypeStruct((H, E), jnp.float32),
            jax.ShapeDtypeStruct((H, 8), jnp.float32),
        ],
    )(z1, g1.reshape(1, H), mu1.reshape(1, H), den1.reshape(1, H),
      be1.reshape(1, H), w2, b2.reshape(H, 1))
    return z2t, st[:, 0], st[:, 1]


def _edge_conv(x, src, dst, W1, b1, g1, be1, W2, b2, g2, be2):
    N, F = x.shape
    E = src.shape[0]
    xd, xs = _gather2_sc(x, src, dst)
    z1, s1, ss1 = _mm1(xd, xs, W1.T, b1)
    mu1 = s1 / E
    var1 = ss1 / E - mu1 * mu1
    den1 = jnp.sqrt(var1 + _EPS)
    z2t, s2, ss2 = _mm2(z1, g1, mu1, den1, be1, W2, b2)
    mu2 = s2 / E
    var2 = ss2 / E - mu2 * mu2
    smax = _segmax_sc(z2t, dst, N).T
    out = g2 * (smax - mu2) / jnp.sqrt(var2 + _EPS) + be2
    return jnp.where(jnp.isfinite(smax), out, 0.0)


def _bn_dense(h, W, b, g, be):
    h = h @ W.T + b
    h = jax.nn.relu(h)
    mu = jnp.mean(h, axis=0)
    var = jnp.mean((h - mu) ** 2, axis=0)
    return g * (h - mu) / jnp.sqrt(var + _EPS) + be


def kernel(x, edge_index, batch, params):
    p = params
    src, dst = edge_index[0], edge_index[1]
    x1 = _edge_conv(x, src, dst, p["c1_w1"], p["c1_b1"], p["c1_g1"], p["c1_be1"],
                    p["c1_w2"], p["c1_b2"], p["c1_g2"], p["c1_be2"])
    x2 = _edge_conv(x1, src, dst, p["c2_w1"], p["c2_b1"], p["c2_g1"], p["c2_be1"],
                    p["c2_w2"], p["c2_b2"], p["c2_g2"], p["c2_be2"])
    h = _bn_dense(jnp.concatenate([x1, x2], axis=1),
                  p["lin_w"], p["lin_b"], p["lin_g"], p["lin_be"])
    G = 16
    sums = jax.ops.segment_sum(h, batch, num_segments=G)
    cnt = jax.ops.segment_sum(jnp.ones((h.shape[0],), h.dtype), batch, num_segments=G)
    pooled = sums / jnp.maximum(cnt, 1.0)[:, None]
    m = _bn_dense(pooled, p["m1_w"], p["m1_b"], p["m1_g"], p["m1_be"])
    m = _bn_dense(m, p["m2_w"], p["m2_b"], p["m2_g"], p["m2_be"])
    return m @ p["out_w"].T + p["out_b"]


# speedup vs baseline: 1.1286x; 1.1286x over previous
import functools
import jax
import jax.numpy as jnp
from jax import lax
from jax.experimental import pallas as pl
from jax.experimental.pallas import tpu as pltpu
from jax.experimental.pallas import tpu_sc as plsc

_EPS = 1e-5
_BE = 6400
_CB = 1280


def _segmax_sc(z2t, dst, n_nodes):
    H, E = z2t.shape
    e_half = E // 2
    n_chunks = e_half // _CB
    gpairs = _CB // 32
    mesh = plsc.VectorSubcoreMesh(core_axis_name="c", subcore_axis_name="s")

    @functools.partial(
        pl.kernel,
        mesh=mesh,
        out_type=jax.ShapeDtypeStruct((2 * H, n_nodes), jnp.float32),
        compiler_params=pltpu.CompilerParams(needs_layout_passes=False),
        scratch_types=[
            pltpu.VMEM((8, n_nodes), jnp.float32),
            pltpu.VMEM((2, _CB), jnp.int32),
            pltpu.VMEM((2, 8, _CB), jnp.float32),
            pltpu.VMEM((n_nodes,), jnp.float32),
            pltpu.SemaphoreType.DMA,
            pltpu.SemaphoreType.DMA,
        ],
    )
    def segmax(z2t_hbm, dst_hbm, out_hbm, acc_v, idx_v, buf_v, arb_v,
               sem0, sem1):
        half = lax.axis_index("c")
        r0 = pl.multiple_of(lax.axis_index("s") * 8, 8)
        e0 = pl.multiple_of(half * e_half, 128)
        lanes0 = lax.iota(jnp.int32, 16).astype(jnp.float32)
        lanes1 = lanes0 + 16.0
        full = jnp.full((16,), True)
        neg_inf = jnp.full((16,), -jnp.inf, jnp.float32)
        sems = (sem0, sem1)

        def start(ch, b):
            cb = pl.multiple_of(e0 + ch * _CB, 128)
            pltpu.async_copy(dst_hbm.at[pl.ds(cb, _CB)], idx_v.at[b], sems[b])
            pltpu.async_copy(z2t_hbm.at[pl.ds(r0, 8), pl.ds(cb, _CB)],
                             buf_v.at[b], sems[b])

        def drain(b):
            pltpu.make_async_copy(dst_hbm.at[pl.ds(0, _CB)], idx_v.at[b],
                                  sems[b]).wait()
            pltpu.make_async_copy(z2t_hbm.at[pl.ds(0, 8), pl.ds(0, _CB)],
                                  buf_v.at[b], sems[b]).wait()

        start(0, 0)
        start(1, 1)

        def init_body(i, _):
            for c in range(8):
                acc_v[c, pl.ds(i * 16, 16)] = neg_inf
            return 0
        lax.fori_loop(0, n_nodes // 16, init_body, 0)

        def upd8(idx, b, g, mask):
            cvecs = [jnp.full((16,), c, jnp.int32) for c in range(8)]
            curs = [plsc.load_gather(acc_v, [cvecs[c], idx])
                    for c in range(8)]
            vs = [buf_v[b, c, pl.ds(g * 16, 16)] for c in range(8)]
            ms = [jnp.maximum(curs[c], vs[c]) for c in range(8)]
            for c in range(8):
                plsc.store_scatter(acc_v, [cvecs[c], idx], ms[c],
                                   mask=mask)

        def process(b):
            def gp_body(g, _):
                ia = idx_v[b, pl.ds(g * 32, 16)]
                ib = idx_v[b, pl.ds(g * 32 + 16, 16)]
                plsc.store_scatter(arb_v, [ia], lanes0, mask=full)
                plsc.store_scatter(arb_v, [ib], lanes1, mask=full)
                won_a = plsc.load_gather(arb_v, [ia]) == lanes0
                won_b = plsc.load_gather(arb_v, [ib]) == lanes1
                upd8(ia, b, 2 * g, won_a)
                upd8(ib, b, 2 * g + 1, won_b)

                def any_losers(carry):
                    la, lb = carry
                    return jnp.any(la) | jnp.any(lb)

                def retry(carry):
                    la, lb = carry
                    plsc.store_scatter(arb_v, [ia], lanes0, mask=la)
                    plsc.store_scatter(arb_v, [ib], lanes1, mask=lb)
                    wa = (plsc.load_gather(arb_v, [ia]) == lanes0) & la
                    wb = (plsc.load_gather(arb_v, [ib]) == lanes1) & lb
                    upd8(ia, b, 2 * g, wa)
                    upd8(ib, b, 2 * g + 1, wb)
                    return (la & jnp.logical_not(wa),
                            lb & jnp.logical_not(wb))

                lax.while_loop(any_losers, retry,
                               (jnp.logical_not(won_a),
                                jnp.logical_not(won_b)))
                return 0
            lax.fori_loop(0, gpairs, gp_body, 0)

        def pair_body(j, _):
            drain(0)
            process(0)

            @pl.when(2 * j + 2 < n_chunks)
            def _():
                start(2 * j + 2, 0)
            drain(1)
            process(1)

            @pl.when(2 * j + 3 < n_chunks)
            def _():
                start(2 * j + 3, 1)
            return 0
        lax.fori_loop(0, n_chunks // 2, pair_body, 0)
        if n_chunks % 2:
            drain(0)
            process(0)
        out_r = pl.multiple_of(half * H + r0, 8)
        pltpu.sync_copy(acc_v, out_hbm.at[pl.ds(out_r, 8), :])

    part = segmax(z2t, dst)
    return jnp.maximum(part[:H], part[H:])


_CG = 640


def _gather2_sc(x, src, dst):
    N, F = x.shape
    E = src.shape[0]
    n_chunks = E // _CG
    mesh = plsc.VectorSubcoreMesh(core_axis_name="c", subcore_axis_name="s")

    @functools.partial(
        pl.kernel,
        mesh=mesh,
        out_type=[jax.ShapeDtypeStruct((E, F), jnp.float32),
                  jax.ShapeDtypeStruct((E, F), jnp.float32)],
        compiler_params=pltpu.CompilerParams(needs_layout_passes=False),
        scratch_types=[
            pltpu.VMEM((_CG,), jnp.int32),
            pltpu.VMEM((_CG, 128), jnp.float32),
            pltpu.SemaphoreType.DMA,
        ],
    )
    def gather2(x_hbm, src_hbm, dst_hbm, xd_hbm, xs_hbm, idx_v, buf_v, sem):
        wid = lax.axis_index("s") * 2 + lax.axis_index("c")
        per_w = (n_chunks + 31) // 32

        def run(idx_hbm, out_hbm):
            def chunk_body(j, _):
                ch = wid + j * 32

                @pl.when(ch < n_chunks)
                def _():
                    cb = pl.multiple_of(ch * _CG, 128)
                    pltpu.sync_copy(idx_hbm.at[pl.ds(cb, _CG)], idx_v)
                    copies = [
                        pltpu.async_copy(
                            x_hbm.at[idx_v.at[pl.ds(k * 128, 128)]],
                            buf_v.at[pl.ds(k * 128, 128)], sem)
                        for k in range(_CG // 128)
                    ]
                    for c in copies:
                        c.wait()
                    pltpu.sync_copy(buf_v, out_hbm.at[pl.ds(cb, _CG), :])
                return 0
            lax.fori_loop(0, per_w, chunk_body, 0)

        run(dst_hbm, xd_hbm)
        run(src_hbm, xs_hbm)

    return gather2(x, src, dst)


def _mm1_body(xd_ref, xs_ref, w_ref, b_ref, z1_ref, st_ref):
    xd = xd_ref[...]
    m = jnp.concatenate([xd, xs_ref[...] - xd], axis=1)
    z1 = jnp.dot(m, w_ref[...], preferred_element_type=jnp.float32)
    z1 = jnp.maximum(z1 + b_ref[...], 0.0)
    z1_ref[...] = z1
    @pl.when(pl.program_id(0) == 0)
    def _():
        st_ref[...] = jnp.zeros_like(st_ref)
    st_ref[0:1, :] += jnp.sum(z1, axis=0, keepdims=True)
    st_ref[1:2, :] += jnp.sum(z1 * z1, axis=0, keepdims=True)


def _mm1(xd, xs, w1_t, b1):
    E, F = xd.shape
    H = w1_t.shape[1]
    z1, st = pl.pallas_call(
        _mm1_body,
        grid=(E // _BE,),
        in_specs=[
            pl.BlockSpec((_BE, F), lambda i: (i, 0)),
            pl.BlockSpec((_BE, F), lambda i: (i, 0)),
            pl.BlockSpec((2 * F, H), lambda i: (0, 0)),
            pl.BlockSpec((1, H), lambda i: (0, 0)),
        ],
        out_specs=[
            pl.BlockSpec((_BE, H), lambda i: (i, 0)),
            pl.BlockSpec((8, H), lambda i: (0, 0)),
        ],
        out_shape=[
            jax.ShapeDtypeStruct((E, H), jnp.float32),
            jax.ShapeDtypeStruct((8, H), jnp.float32),
        ],
    )(xd, xs, w1_t, b1.reshape(1, H))
    return z1, st[0], st[1]


def _mm2_body(z1_ref, g_ref, mu_ref, den_ref, be_ref, w_ref, b_ref, z2t_ref, st_ref):
    h1 = g_ref[...] * (z1_ref[...] - mu_ref[...]) / den_ref[...] + be_ref[...]
    z2t = lax.dot_general(w_ref[...], h1, (((1,), (1,)), ((), ())),
                          preferred_element_type=jnp.float32)
    z2t = jnp.maximum(z2t + b_ref[...], 0.0)
    z2t_ref[...] = z2t
    @pl.when(pl.program_id(0) == 0)
    def _():
        st_ref[...] = jnp.zeros_like(st_ref)
    st_ref[:, 0:1] += jnp.sum(z2t, axis=1, keepdims=True)
    st_ref[:, 1:2] += jnp.sum(z2t * z2t, axis=1, keepdims=True)


def _mm2(z1, g1, mu1, den1, be1, w2, b2):
    E, H = z1.shape
    z2t, st = pl.pallas_call(
        _mm2_body,
        grid=(E // _BE,),
        in_specs=[
            pl.BlockSpec((_BE, H), lambda i: (i, 0)),
            pl.BlockSpec((1, H), lambda i: (0, 0)),
            pl.BlockSpec((1, H), lambda i: (0, 0)),
            pl.BlockSpec((1, H), lambda i: (0, 0)),
            pl.BlockSpec((1, H), lambda i: (0, 0)),
            pl.BlockSpec((H, H), lambda i: (0, 0)),
            pl.BlockSpec((H, 1), lambda i: (0, 0)),
        ],
        out_specs=[
            pl.BlockSpec((H, _BE), lambda i: (0, i)),
            pl.BlockSpec((H, 8), lambda i: (0, 0)),
        ],
        out_shape=[
            jax.ShapeDtypeStruct((H, E), jnp.float32),
            jax.ShapeDtypeStruct((H, 8), jnp.float32),
        ],
    )(z1, g1.reshape(1, H), mu1.reshape(1, H), den1.reshape(1, H),
      be1.reshape(1, H), w2, b2.reshape(H, 1))
    return z2t, st[:, 0], st[:, 1]


def _edge_conv(x, src, dst, W1, b1, g1, be1, W2, b2, g2, be2):
    N, F = x.shape
    E = src.shape[0]
    xd, xs = _gather2_sc(x, src, dst)
    z1, s1, ss1 = _mm1(xd, xs, W1.T, b1)
    mu1 = s1 / E
    var1 = ss1 / E - mu1 * mu1
    den1 = jnp.sqrt(var1 + _EPS)
    z2t, s2, ss2 = _mm2(z1, g1, mu1, den1, be1, W2, b2)
    mu2 = s2 / E
    var2 = ss2 / E - mu2 * mu2
    smax = _segmax_sc(z2t, dst, N).T
    out = g2 * (smax - mu2) / jnp.sqrt(var2 + _EPS) + be2
    return jnp.where(jnp.isfinite(smax), out, 0.0)


_BN = 2000


def _head_body(x1_ref, x2_ref, bt_ref, w_ref, b_ref, st_ref, ps_ref):
    h = jnp.concatenate([x1_ref[...], x2_ref[...]], axis=1)
    h = jnp.dot(h, w_ref[...], preferred_element_type=jnp.float32)
    h = jnp.maximum(h + b_ref[...], 0.0)
    onehot_t = (lax.broadcasted_iota(jnp.int32, (_BN, 16), 1)
                == bt_ref[...]).astype(jnp.float32)
    @pl.when(pl.program_id(0) == 0)
    def _():
        st_ref[...] = jnp.zeros_like(st_ref)
        ps_ref[...] = jnp.zeros_like(ps_ref)
    st_ref[0:1, :] += jnp.sum(h, axis=0, keepdims=True)
    st_ref[1:2, :] += jnp.sum(h * h, axis=0, keepdims=True)
    ps_ref[...] += lax.dot_general(onehot_t, h, (((0,), (0,)), ((), ())),
                                   preferred_element_type=jnp.float32,
                                   precision=lax.Precision.HIGHEST)


def _head_pool(x1, x2, batch, lin_wt, lin_b):
    N, F = x1.shape
    M = lin_wt.shape[1]
    st, ps = pl.pallas_call(
        _head_body,
        grid=(N // _BN,),
        in_specs=[
            pl.BlockSpec((_BN, F), lambda i: (i, 0)),
            pl.BlockSpec((_BN, F), lambda i: (i, 0)),
            pl.BlockSpec((_BN, 1), lambda i: (i, 0)),
            pl.BlockSpec((2 * F, M), lambda i: (0, 0)),
            pl.BlockSpec((1, M), lambda i: (0, 0)),
        ],
        out_specs=[
            pl.BlockSpec((8, M), lambda i: (0, 0)),
            pl.BlockSpec((16, M), lambda i: (0, 0)),
        ],
        out_shape=[
            jax.ShapeDtypeStruct((8, M), jnp.float32),
            jax.ShapeDtypeStruct((16, M), jnp.float32),
        ],
    )(x1, x2, batch.reshape(N, 1), lin_wt, lin_b.reshape(1, M))
    return st[0], st[1], ps


def _bn_dense(h, W, b, g, be):
    h = h @ W.T + b
    h = jax.nn.relu(h)
    mu = jnp.mean(h, axis=0)
    var = jnp.mean((h - mu) ** 2, axis=0)
    return g * (h - mu) / jnp.sqrt(var + _EPS) + be


def kernel(x, edge_index, batch, params):
    p = params
    src, dst = edge_index[0], edge_index[1]
    x1 = _edge_conv(x, src, dst, p["c1_w1"], p["c1_b1"], p["c1_g1"], p["c1_be1"],
                    p["c1_w2"], p["c1_b2"], p["c1_g2"], p["c1_be2"])
    x2 = _edge_conv(x1, src, dst, p["c2_w1"], p["c2_b1"], p["c2_g1"], p["c2_be1"],
                    p["c2_w2"], p["c2_b2"], p["c2_g2"], p["c2_be2"])
    G = 16
    N = x1.shape[0]
    hs, hss, psum = _head_pool(x1, x2, batch, p["lin_w"].T, p["lin_b"])
    mu = hs / N
    var = hss / N - mu * mu
    den = jnp.sqrt(var + _EPS)
    edges = jnp.searchsorted(batch, jnp.arange(G + 1, dtype=batch.dtype))
    cnt = (edges[1:] - edges[:-1]).astype(jnp.float32)
    pmean = psum / jnp.maximum(cnt, 1.0)[:, None]
    pooled = jnp.where(cnt[:, None] > 0,
                       p["lin_g"] * (pmean - mu) / den + p["lin_be"], 0.0)
    m = _bn_dense(pooled, p["m1_w"], p["m1_b"], p["m1_g"], p["m1_be"])
    m = _bn_dense(m, p["m2_w"], p["m2_b"], p["m2_g"], p["m2_be"])
    return m @ p["out_w"].T + p["out_b"]
